# Initial kernel scaffold; baseline (speedup 1.0000x reference)
import jax
import jax.numpy as jnp
from jax.experimental import pallas as pl


def kernel(x, edge_index, W_in, b_in, W1, b1, W2, b2, W_out, b_out):
    raise NotImplementedError("write your pallas kernel here")


# speedup vs baseline: 5.2371x; 5.2371x over previous
import functools

import jax
import jax.numpy as jnp
from jax import lax
from jax.experimental import pallas as pl
from jax.experimental.pallas import tpu as pltpu
from jax.experimental.pallas import tpu_sc as plsc

N = 10000
E = 320000
D = 128
H = 128
C = 40

NC = 2
NS = 16
NW = NC * NS
KC = 64
IB = 8
NBLK = -(-E // (NW * KC * IB))
NCHUNK = NBLK * IB
E_PAD = NW * NCHUNK * KC
N_ACC = 10240
ROWS_PER_TILE = N_ACC // NS
WB = ROWS_PER_TILE // KC

_mesh = plsc.VectorSubcoreMesh(core_axis_name="c", subcore_axis_name="s")


def _fill_const(ref, rows, width, value):
    vec = jnp.full((16,), value, jnp.float32)

    def body(i, _):
        r = i // (width // 16)
        c = (i % (width // 16)) * 16
        ref[r, pl.ds(c, 16)] = vec
        return 0

    lax.fori_loop(0, rows * (width // 16), body, 0)


@functools.partial(
    pl.kernel,
    out_type=jax.ShapeDtypeStruct((NC, N_ACC, H), jnp.float32),
    mesh=_mesh,
    scratch_types=[
        pltpu.VMEM((2, IB, KC), jnp.int32),
        pltpu.VMEM((2, IB, KC), jnp.int32),
        pltpu.VMEM((KC, H), jnp.float32),
        pltpu.VMEM((KC, H), jnp.float32),
        pltpu.VMEM_SHARED((N_ACC, H), jnp.float32),
        pltpu.SemaphoreType.DMA,
        pltpu.SemaphoreType.DMA,
    ],
)
def _agg_sc(z_hbm, col_hbm, row_hbm, out_hbm, col_v, row_v, buf_a, buf_b, acc,
            sem_a, sem_b):
    cid = lax.axis_index("c")
    sid = lax.axis_index("s")
    wid = sid * NC + cid
    base = sid * ROWS_PER_TILE

    _fill_const(buf_a, KC, H, 0.0)
    for k in range(WB):
        pltpu.sync_copy(buf_a, acc.at[pl.ds(base + k * KC, KC)])
    pltpu.sync_copy(col_hbm.at[wid, pl.ds(0, IB)], col_v.at[0])
    pltpu.sync_copy(row_hbm.at[wid, pl.ds(0, IB)], row_v.at[0])
    if NBLK > 1:
        pltpu.sync_copy(col_hbm.at[wid, pl.ds(IB, IB)], col_v.at[1])
        pltpu.sync_copy(row_hbm.at[wid, pl.ds(IB, IB)], row_v.at[1])
    plsc.subcore_barrier()

    pltpu.async_copy(z_hbm.at[col_v.at[0, 0]], buf_a, sem_a)

    def step(j, cur, nxt, sem_cur, sem_nxt):
        b = j // IB
        r = j % IB
        @pl.when((r == 0) & (j >= IB) & (b + 1 < NBLK))
        def _():
            slot = (b + 1) % 2
            pltpu.sync_copy(col_hbm.at[wid, pl.ds((b + 1) * IB, IB)],
                            col_v.at[slot])
            pltpu.sync_copy(row_hbm.at[wid, pl.ds((b + 1) * IB, IB)],
                            row_v.at[slot])

        pltpu.make_async_copy(z_hbm.at[col_v.at[b % 2, r]], cur, sem_cur).wait()

        @pl.when(j + 1 < NCHUNK)
        def _():
            jn = j + 1
            pltpu.async_copy(z_hbm.at[col_v.at[(jn // IB) % 2, jn % IB]],
                             nxt, sem_nxt)

        pltpu.sync_copy(cur, acc.at[row_v.at[b % 2, r]], add=True)

    def body(j, _):
        @pl.when(j % 2 == 0)
        def _():
            step(j, buf_a, buf_b, sem_a, sem_b)

        @pl.when(j % 2 == 1)
        def _():
            step(j, buf_b, buf_a, sem_b, sem_a)

        return 0

    lax.fori_loop(0, NCHUNK, body, 0)
    plsc.subcore_barrier()

    for k in range(WB):
        pltpu.sync_copy(acc.at[pl.ds(base + k * KC, KC)], buf_a)
        pltpu.sync_copy(buf_a, out_hbm.at[cid, pl.ds(base + k * KC, KC)])


BR = 2000


def _dinv(d0, d1):
    deg = d0[:, 0:1] + d1[:, 0:1] + 1.0
    return lax.rsqrt(jnp.maximum(deg, 1e-12))


def _tc_in_body(x_ref, w_ref, d0_ref, d1_ref, z_ref):
    dinv = _dinv(d0_ref[...], d1_ref[...])
    z_ref[...] = dinv * jnp.dot(x_ref[...], w_ref[...],
                                preferred_element_type=jnp.float32)


def _tc_mid_body(d0_ref, d1_ref, a0_ref, a1_ref, z_ref, b_ref, w_ref,
                 h_ref, zn_ref):
    dinv = _dinv(d0_ref[...], d1_ref[...])
    h = jnp.maximum(dinv * (a0_ref[...] + a1_ref[...] + z_ref[...])
                    + b_ref[...], 0.0)
    h_ref[...] = h
    zn_ref[...] = dinv * jnp.dot(h, w_ref[...],
                                 preferred_element_type=jnp.float32)


def _tc_out_body(d0_ref, d1_ref, a0_ref, a1_ref, z_ref, b_ref,
                 h1_ref, h2_ref, wo_ref, bo_ref, o_ref):
    dinv = _dinv(d0_ref[...], d1_ref[...])
    h3 = jnp.maximum(dinv * (a0_ref[...] + a1_ref[...] + z_ref[...])
                     + b_ref[...], 0.0)
    acc = jnp.dot(h1_ref[...], wo_ref[0], preferred_element_type=jnp.float32)
    acc += jnp.dot(h2_ref[...], wo_ref[1], preferred_element_type=jnp.float32)
    acc += jnp.dot(h3, wo_ref[2], preferred_element_type=jnp.float32)
    o_ref[...] = acc + bo_ref[...]


def _row_spec(width):
    return pl.BlockSpec((BR, width), lambda i: (i, 0))


def _full_spec(shape):
    return pl.BlockSpec(shape, lambda i: tuple(0 for _ in shape))


def _tc_in(x, w, d0, d1):
    return pl.pallas_call(
        _tc_in_body,
        grid=(N // BR,),
        in_specs=[_row_spec(D), _full_spec((D, H)), _row_spec(16), _row_spec(16)],
        out_specs=_row_spec(H),
        out_shape=jax.ShapeDtypeStruct((N, H), jnp.float32),
    )(x, w, d0, d1)


def _tc_mid(d0, d1, a0, a1, z, b, w):
    return pl.pallas_call(
        _tc_mid_body,
        grid=(N // BR,),
        in_specs=[_row_spec(16), _row_spec(16), _row_spec(H), _row_spec(H),
                  _row_spec(H), _full_spec((1, H)), _full_spec((H, H))],
        out_specs=[_row_spec(H), _row_spec(H)],
        out_shape=[jax.ShapeDtypeStruct((N, H), jnp.float32),
                   jax.ShapeDtypeStruct((N, H), jnp.float32)],
    )(d0, d1, a0, a1, z, b, w)


def _tc_out(d0, d1, a0, a1, z, b, h1, h2, wo, bo):
    return pl.pallas_call(
        _tc_out_body,
        grid=(N // BR,),
        in_specs=[_row_spec(16), _row_spec(16), _row_spec(H), _row_spec(H),
                  _row_spec(H), _full_spec((1, H)), _row_spec(H), _row_spec(H),
                  _full_spec((3, H, 128)), _full_spec((1, 128))],
        out_specs=_row_spec(128),
        out_shape=jax.ShapeDtypeStruct((N, 128), jnp.float32),
    )(d0, d1, a0, a1, z, b, h1, h2, wo, bo)


def kernel(x, edge_index, W_in, b_in, W1, b1, W2, b2, W_out, b_out):
    row = edge_index[0]
    col = edge_index[1]
    pad = E_PAD - E
    row_p = jnp.concatenate([row, jnp.full((pad,), N, jnp.int32)])
    col_p = jnp.concatenate([col, jnp.zeros((pad,), jnp.int32)])
    row3 = row_p.reshape(NW, NCHUNK, KC)
    col3 = col_p.reshape(NW, NCHUNK, KC)

    degp = _agg_sc(jnp.ones((N, H), jnp.float32), col3, row3)
    d0, d1 = degp[0, :, :16], degp[1, :, :16]

    z1 = _tc_in(x, W_in, d0[:N], d1[:N])
    a = _agg_sc(z1, col3, row3)
    h1, z2 = _tc_mid(d0[:N], d1[:N], a[0, :N], a[1, :N], z1,
                     b_in.reshape(1, H), W1)
    a = _agg_sc(z2, col3, row3)
    h2, z3 = _tc_mid(d0[:N], d1[:N], a[0, :N], a[1, :N], z2,
                     b1.reshape(1, H), W2)
    a = _agg_sc(z3, col3, row3)

    wo = jnp.zeros((3, H, 128), jnp.float32).at[:, :, :C].set(
        W_out.reshape(3, H, C))
    bo = jnp.zeros((1, 128), jnp.float32).at[0, :C].set(b_out)
    out = _tc_out(d0[:N], d1[:N], a[0, :N], a[1, :N], z3,
                  b2.reshape(1, H), h1, h2, wo, bo)
    return out[:, :C]


# speedup vs baseline: 20.8688x; 3.9848x over previous
import functools

import jax
import jax.numpy as jnp
from jax import lax
from jax.experimental import pallas as pl
from jax.experimental.pallas import tpu as pltpu
from jax.experimental.pallas import tpu_sc as plsc

N = 10000
E = 320000
D = 128
H = 128
C = 40

NC = 2
NS = 16
NW = NC * NS
KC = 64
IB = 8
NBLK = -(-E // (NW * KC * IB))
NCHUNK = NBLK * IB
E_PAD = NW * NCHUNK * KC
N_ACC = 10240
ROWS_PER_TILE = N_ACC // NS
WB = ROWS_PER_TILE // KC

_mesh = plsc.VectorSubcoreMesh(core_axis_name="c", subcore_axis_name="s")


def _fill_const(ref, rows, width, value):
    vec = jnp.full((16,), value, jnp.float32)

    def body(i, _):
        r = i // (width // 16)
        c = (i % (width // 16)) * 16
        ref[r, pl.ds(c, 16)] = vec
        return 0

    lax.fori_loop(0, rows * (width // 16), body, 0)


NBUF = 4


@functools.partial(
    pl.kernel,
    out_type=jax.ShapeDtypeStruct((NC, N_ACC, H), jnp.float32),
    mesh=_mesh,
    scratch_types=[
        pltpu.VMEM((2, IB, KC), jnp.int32),
        pltpu.VMEM((2, IB, KC), jnp.int32),
    ] + [pltpu.VMEM((KC, H), jnp.float32) for _ in range(NBUF)]
    + [pltpu.VMEM_SHARED((N_ACC, H), jnp.float32)]
    + [pltpu.SemaphoreType.DMA for _ in range(NBUF)],
)
def _agg_sc(z_hbm, col_hbm, row_hbm, out_hbm, col_v, row_v, *rest):
    bufs = rest[:NBUF]
    acc = rest[NBUF]
    sems = rest[NBUF + 1:]
    cid = lax.axis_index("c")
    sid = lax.axis_index("s")
    wid = sid * NC + cid
    base = sid * ROWS_PER_TILE

    _fill_const(bufs[0], KC, H, 0.0)
    for k in range(WB):
        pltpu.sync_copy(bufs[0], acc.at[pl.ds(base + k * KC, KC)])
    pltpu.sync_copy(col_hbm.at[wid, pl.ds(0, IB)], col_v.at[0])
    pltpu.sync_copy(row_hbm.at[wid, pl.ds(0, IB)], row_v.at[0])
    pltpu.sync_copy(col_hbm.at[wid, pl.ds(IB, IB)], col_v.at[1])
    pltpu.sync_copy(row_hbm.at[wid, pl.ds(IB, IB)], row_v.at[1])
    plsc.subcore_barrier()

    for p in range(NBUF - 1):
        pltpu.async_copy(z_hbm.at[col_v.at[(p // IB) % 2, p % IB]],
                         bufs[p], sems[p])

    def step(j, slot):
        b = j // IB
        r = j % IB
        @pl.when((r == 0) & (j >= IB) & (b + 1 < NBLK))
        def _():
            sl = (b + 1) % 2
            pltpu.sync_copy(col_hbm.at[wid, pl.ds((b + 1) * IB, IB)],
                            col_v.at[sl])
            pltpu.sync_copy(row_hbm.at[wid, pl.ds((b + 1) * IB, IB)],
                            row_v.at[sl])

        jn = j + NBUF - 1
        nslot = (slot + NBUF - 1) % NBUF

        @pl.when(jn < NCHUNK)
        def _():
            pltpu.async_copy(z_hbm.at[col_v.at[(jn // IB) % 2, jn % IB]],
                             bufs[nslot], sems[nslot])

        pltpu.make_async_copy(z_hbm.at[col_v.at[b % 2, r]], bufs[slot],
                              sems[slot]).wait()
        pltpu.sync_copy(bufs[slot], acc.at[row_v.at[b % 2, r]], add=True)

    def body(j, _):
        for s in range(NBUF):
            @pl.when(j % NBUF == s)
            def _(s=s):
                step(j, s)
        return 0

    lax.fori_loop(0, NCHUNK, body, 0)
    plsc.subcore_barrier()

    for k in range(WB):
        pltpu.sync_copy(acc.at[pl.ds(base + k * KC, KC)], bufs[0])
        pltpu.sync_copy(bufs[0], out_hbm.at[cid, pl.ds(base + k * KC, KC)])


BR = 2000


def _dinv(d0, d1):
    deg = d0[:, 0:1] + d1[:, 0:1] + 1.0
    return lax.rsqrt(jnp.maximum(deg, 1e-12))


def _tc_in_body(x_ref, w_ref, d0_ref, d1_ref, z_ref):
    dinv = _dinv(d0_ref[...], d1_ref[...])
    z_ref[...] = dinv * jnp.dot(x_ref[...], w_ref[...],
                                preferred_element_type=jnp.float32)


def _tc_mid_body(d0_ref, d1_ref, a0_ref, a1_ref, z_ref, b_ref, w_ref,
                 h_ref, zn_ref):
    dinv = _dinv(d0_ref[...], d1_ref[...])
    h = jnp.maximum(dinv * (a0_ref[...] + a1_ref[...] + z_ref[...])
                    + b_ref[...], 0.0)
    h_ref[...] = h
    zn_ref[...] = dinv * jnp.dot(h, w_ref[...],
                                 preferred_element_type=jnp.float32)


def _tc_out_body(d0_ref, d1_ref, a0_ref, a1_ref, z_ref, b_ref,
                 h1_ref, h2_ref, wo_ref, bo_ref, o_ref):
    dinv = _dinv(d0_ref[...], d1_ref[...])
    h3 = jnp.maximum(dinv * (a0_ref[...] + a1_ref[...] + z_ref[...])
                     + b_ref[...], 0.0)
    acc = jnp.dot(h1_ref[...], wo_ref[0], preferred_element_type=jnp.float32)
    acc += jnp.dot(h2_ref[...], wo_ref[1], preferred_element_type=jnp.float32)
    acc += jnp.dot(h3, wo_ref[2], preferred_element_type=jnp.float32)
    o_ref[...] = acc + bo_ref[...]


def _row_spec(width):
    return pl.BlockSpec((BR, width), lambda i: (i, 0))


def _full_spec(shape):
    return pl.BlockSpec(shape, lambda i: tuple(0 for _ in shape))


def _tc_in(x, w, d0, d1):
    return pl.pallas_call(
        _tc_in_body,
        grid=(N // BR,),
        in_specs=[_row_spec(D), _full_spec((D, H)), _row_spec(16), _row_spec(16)],
        out_specs=_row_spec(H),
        out_shape=jax.ShapeDtypeStruct((N, H), jnp.float32),
    )(x, w, d0, d1)


def _tc_mid(d0, d1, a0, a1, z, b, w):
    return pl.pallas_call(
        _tc_mid_body,
        grid=(N // BR,),
        in_specs=[_row_spec(16), _row_spec(16), _row_spec(H), _row_spec(H),
                  _row_spec(H), _full_spec((1, H)), _full_spec((H, H))],
        out_specs=[_row_spec(H), _row_spec(H)],
        out_shape=[jax.ShapeDtypeStruct((N, H), jnp.float32),
                   jax.ShapeDtypeStruct((N, H), jnp.float32)],
    )(d0, d1, a0, a1, z, b, w)


def _tc_out(d0, d1, a0, a1, z, b, h1, h2, wo, bo):
    return pl.pallas_call(
        _tc_out_body,
        grid=(N // BR,),
        in_specs=[_row_spec(16), _row_spec(16), _row_spec(H), _row_spec(H),
                  _row_spec(H), _full_spec((1, H)), _row_spec(H), _row_spec(H),
                  _full_spec((3, H, 128)), _full_spec((1, 128))],
        out_specs=_row_spec(128),
        out_shape=jax.ShapeDtypeStruct((N, 128), jnp.float32),
    )(d0, d1, a0, a1, z, b, h1, h2, wo, bo)


def kernel(x, edge_index, W_in, b_in, W1, b1, W2, b2, W_out, b_out):
    row = edge_index[0]
    col = edge_index[1]
    pad = E_PAD - E
    ar = jnp.arange(pad, dtype=jnp.int32)
    row_p = jnp.concatenate([row, N + ar % (N_ACC - N)])
    col_p = jnp.concatenate([col, ar % N])
    row3 = row_p.reshape(NW, NCHUNK, KC)
    col3 = col_p.reshape(NW, NCHUNK, KC)

    degp = _agg_sc(jnp.ones((N, H), jnp.float32), col3, row3)
    d0, d1 = degp[0, :, :16], degp[1, :, :16]

    z1 = _tc_in(x, W_in, d0[:N], d1[:N])
    a = _agg_sc(z1, col3, row3)
    h1, z2 = _tc_mid(d0[:N], d1[:N], a[0, :N], a[1, :N], z1,
                     b_in.reshape(1, H), W1)
    a = _agg_sc(z2, col3, row3)
    h2, z3 = _tc_mid(d0[:N], d1[:N], a[0, :N], a[1, :N], z2,
                     b1.reshape(1, H), W2)
    a = _agg_sc(z3, col3, row3)

    wo = jnp.zeros((3, H, 128), jnp.float32).at[:, :, :C].set(
        W_out.reshape(3, H, C))
    bo = jnp.zeros((1, 128), jnp.float32).at[0, :C].set(b_out)
    out = _tc_out(d0[:N], d1[:N], a[0, :N], a[1, :N], z3,
                  b2.reshape(1, H), h1, h2, wo, bo)
    return out[:, :C]


# speedup vs baseline: 22.0813x; 1.0581x over previous
import functools

import jax
import jax.numpy as jnp
from jax import lax
from jax.experimental import pallas as pl
from jax.experimental.pallas import tpu as pltpu
from jax.experimental.pallas import tpu_sc as plsc

N = 10000
E = 320000
D = 128
H = 128
C = 40

NC = 2
NS = 16
NW = NC * NS
KC = 64
IB = 8
NBLK = -(-E // (NW * KC * IB))
NCHUNK = NBLK * IB
E_PAD = NW * NCHUNK * KC
N_ACC = 10240
ROWS_PER_TILE = N_ACC // NS
WB = ROWS_PER_TILE // KC

_mesh = plsc.VectorSubcoreMesh(core_axis_name="c", subcore_axis_name="s")


def _fill_const(ref, rows, width, value):
    vec = jnp.full((16,), value, jnp.float32)

    def body(i, _):
        r = i // (width // 16)
        c = (i % (width // 16)) * 16
        ref[r, pl.ds(c, 16)] = vec
        return 0

    lax.fori_loop(0, rows * (width // 16), body, 0)


NBUF = 4


@functools.partial(
    pl.kernel,
    out_type=jax.ShapeDtypeStruct((NC, N_ACC, H), jnp.float32),
    mesh=_mesh,
    scratch_types=[
        pltpu.VMEM((2, IB, KC), jnp.int32),
        pltpu.VMEM((2, IB, KC), jnp.int32),
    ] + [pltpu.VMEM((KC, H), jnp.float32) for _ in range(NBUF)]
    + [pltpu.VMEM_SHARED((N_ACC, H), jnp.float32)]
    + [pltpu.SemaphoreType.DMA for _ in range(NBUF)],
)
def _agg_sc(z_hbm, col_hbm, row_hbm, out_hbm, col_v, row_v, *rest):
    bufs = rest[:NBUF]
    acc = rest[NBUF]
    sems = rest[NBUF + 1:]
    cid = lax.axis_index("c")
    sid = lax.axis_index("s")
    wid = sid * NC + cid
    base = sid * ROWS_PER_TILE

    _fill_const(bufs[0], KC, H, 0.0)
    for k in range(WB):
        pltpu.sync_copy(bufs[0], acc.at[pl.ds(base + k * KC, KC)])
    pltpu.sync_copy(col_hbm.at[wid, pl.ds(0, IB)], col_v.at[0])
    pltpu.sync_copy(row_hbm.at[wid, pl.ds(0, IB)], row_v.at[0])
    pltpu.sync_copy(col_hbm.at[wid, pl.ds(IB, IB)], col_v.at[1])
    pltpu.sync_copy(row_hbm.at[wid, pl.ds(IB, IB)], row_v.at[1])
    plsc.subcore_barrier()

    for p in range(NBUF - 1):
        pltpu.async_copy(z_hbm.at[col_v.at[(p // IB) % 2, p % IB]],
                         bufs[p], sems[p])

    def step(j, slot):
        b = j // IB
        r = j % IB
        @pl.when((r == 0) & (j >= IB) & (b + 1 < NBLK))
        def _():
            sl = (b + 1) % 2
            pltpu.sync_copy(col_hbm.at[wid, pl.ds((b + 1) * IB, IB)],
                            col_v.at[sl])
            pltpu.sync_copy(row_hbm.at[wid, pl.ds((b + 1) * IB, IB)],
                            row_v.at[sl])

        jn = j + NBUF - 1
        nslot = (slot + NBUF - 1) % NBUF

        @pl.when(jn < NCHUNK)
        def _():
            pltpu.async_copy(z_hbm.at[col_v.at[(jn // IB) % 2, jn % IB]],
                             bufs[nslot], sems[nslot])

        pltpu.make_async_copy(z_hbm.at[col_v.at[b % 2, r]], bufs[slot],
                              sems[slot]).wait()
        pltpu.sync_copy(bufs[slot], acc.at[row_v.at[b % 2, r]], add=True)

    def body(j, _):
        for s in range(NBUF):
            @pl.when(j % NBUF == s)
            def _(s=s):
                step(j, s)
        return 0

    lax.fori_loop(0, NCHUNK, body, 0)
    plsc.subcore_barrier()

    for k in range(WB):
        pltpu.sync_copy(acc.at[pl.ds(base + k * KC, KC)], bufs[0])
        pltpu.sync_copy(bufs[0], out_hbm.at[cid, pl.ds(base + k * KC, KC)])


BR = 2000


def _tc_in_body(x_ref, w_ref, d0_ref, d1_ref, z_ref, dv_ref):
    deg = d0_ref[0, :, 0:1] + d1_ref[0, :, 0:1] + 1.0
    dinv = lax.rsqrt(jnp.maximum(deg, 1e-12))
    dv_ref[...] = jnp.broadcast_to(dinv, dv_ref.shape)
    z_ref[...] = dinv * jnp.dot(x_ref[...], w_ref[...],
                                preferred_element_type=jnp.float32)


def _tc_mid_body(dv_ref, a0_ref, a1_ref, z_ref, b_ref, w_ref,
                 h_ref, zn_ref):
    dinv = dv_ref[:, 0:1]
    h = jnp.maximum(dinv * (a0_ref[0] + a1_ref[0] + z_ref[...])
                    + b_ref[...], 0.0)
    h_ref[...] = h
    zn_ref[...] = dinv * jnp.dot(h, w_ref[...],
                                 preferred_element_type=jnp.float32)


def _tc_out_body(dv_ref, a0_ref, a1_ref, z_ref, b_ref,
                 h1_ref, h2_ref, wo_ref, bo_ref, o_ref):
    dinv = dv_ref[:, 0:1]
    h3 = jnp.maximum(dinv * (a0_ref[0] + a1_ref[0] + z_ref[...])
                     + b_ref[...], 0.0)
    acc = jnp.dot(h1_ref[...], wo_ref[0], preferred_element_type=jnp.float32)
    acc += jnp.dot(h2_ref[...], wo_ref[1], preferred_element_type=jnp.float32)
    acc += jnp.dot(h3, wo_ref[2], preferred_element_type=jnp.float32)
    o_ref[...] = acc + bo_ref[...]


def _row_spec(width):
    return pl.BlockSpec((BR, width), lambda i: (i, 0))


def _part_spec(c, width):
    return pl.BlockSpec((1, BR, width), lambda i, c=c: (c, i, 0))


def _full_spec(shape):
    return pl.BlockSpec(shape, lambda i: tuple(0 for _ in shape))


def _tc_in(x, w, degp):
    return pl.pallas_call(
        _tc_in_body,
        grid=(N // BR,),
        in_specs=[_row_spec(D), _full_spec((D, H)),
                  _part_spec(0, H), _part_spec(1, H)],
        out_specs=[_row_spec(H), _row_spec(8)],
        out_shape=[jax.ShapeDtypeStruct((N, H), jnp.float32),
                   jax.ShapeDtypeStruct((N, 8), jnp.float32)],
    )(x, w, degp, degp)


def _tc_mid(dv, a, z, b, w):
    return pl.pallas_call(
        _tc_mid_body,
        grid=(N // BR,),
        in_specs=[_row_spec(8),
                  _part_spec(0, H), _part_spec(1, H),
                  _row_spec(H), _full_spec((1, H)), _full_spec((H, H))],
        out_specs=[_row_spec(H), _row_spec(H)],
        out_shape=[jax.ShapeDtypeStruct((N, H), jnp.float32),
                   jax.ShapeDtypeStruct((N, H), jnp.float32)],
    )(dv, a, a, z, b, w)


def _tc_out(dv, a, z, b, h1, h2, wo, bo):
    return pl.pallas_call(
        _tc_out_body,
        grid=(N // BR,),
        in_specs=[_row_spec(8),
                  _part_spec(0, H), _part_spec(1, H),
                  _row_spec(H), _full_spec((1, H)), _row_spec(H), _row_spec(H),
                  _full_spec((3, H, 128)), _full_spec((1, 128))],
        out_specs=_row_spec(128),
        out_shape=jax.ShapeDtypeStruct((N, 128), jnp.float32),
    )(dv, a, a, z, b, h1, h2, wo, bo)


def kernel(x, edge_index, W_in, b_in, W1, b1, W2, b2, W_out, b_out):
    row = edge_index[0]
    col = edge_index[1]
    pad = E_PAD - E
    ar = jnp.arange(pad, dtype=jnp.int32)
    row_p = jnp.concatenate([row, N + ar % (N_ACC - N)])
    col_p = jnp.concatenate([col, ar % N])
    row3 = row_p.reshape(NW, NCHUNK, KC)
    col3 = col_p.reshape(NW, NCHUNK, KC)

    degp = _agg_sc(jnp.ones((N, H), jnp.float32), col3, row3)

    z1, dv = _tc_in(x, W_in, degp)
    a = _agg_sc(z1, col3, row3)
    h1, z2 = _tc_mid(dv, a, z1, b_in.reshape(1, H), W1)
    a = _agg_sc(z2, col3, row3)
    h2, z3 = _tc_mid(dv, a, z2, b1.reshape(1, H), W2)
    a = _agg_sc(z3, col3, row3)

    wo = jnp.zeros((3, H, 128), jnp.float32).at[:, :, :C].set(
        W_out.reshape(3, H, C))
    bo = jnp.zeros((1, 128), jnp.float32).at[0, :C].set(b_out)
    out = _tc_out(dv, a, z3, b2.reshape(1, H), h1, h2, wo, bo)
    return out[:, :C]
